# baseline (speedup 1.0000x reference)
import jax
from jax.experimental import pallas as pl
from jax.experimental.pallas import tpu as pltpu

_BLK = 896


def _add_body(x_ref, pe_ref, o_ref):
    o_ref[...] = x_ref[...] + pe_ref[...][None, :, :]


def kernel(x, pe_table):
    B, S, D = x.shape
    n = pl.cdiv(S, _BLK)
    return pl.pallas_call(
        _add_body,
        out_shape=jax.ShapeDtypeStruct((B, S, D), x.dtype),
        grid=(n,),
        in_specs=[
            pl.BlockSpec((B, _BLK, D), lambda i: (0, n - 1 - i, 0)),
            pl.BlockSpec((_BLK, D), lambda i: (n - 1 - i, 0)),
        ],
        out_specs=pl.BlockSpec((B, _BLK, D), lambda i: (0, n - 1 - i, 0)),
        compiler_params=pltpu.CompilerParams(vmem_limit_bytes=67108864),
    )(x, pe_table)


# speedup vs baseline: 1.0923x; 1.0923x over previous
import jax
from jax.experimental import pallas as pl
from jax.experimental.pallas import tpu as pltpu

_BLK = 896


def _add_body(x_ref, pe_ref, o_ref):
    o_ref[...] = x_ref[...] + pe_ref[...][None, :, :]


def kernel(x, pe_table):
    B, S, D = x.shape
    n = pl.cdiv(S, _BLK)
    return pl.pallas_call(
        _add_body,
        out_shape=jax.ShapeDtypeStruct((B, S, D), x.dtype),
        grid=(n,),
        in_specs=[
            pl.BlockSpec((B, _BLK, D), lambda i: (0, i, 0)),
            pl.BlockSpec((_BLK, D), lambda i: (i, 0)),
        ],
        out_specs=pl.BlockSpec((B, _BLK, D), lambda i: (0, i, 0)),
        compiler_params=pltpu.CompilerParams(vmem_limit_bytes=67108864),
    )(x, pe_table)
